# Initial kernel scaffold; baseline (speedup 1.0000x reference)
import jax
import jax.numpy as jnp
from jax.experimental import pallas as pl


def kernel(x, src0, dst0, src1, dst1, W1_0, W2_0, b2_0, gamma0, beta0, W1_1, W2_1, b2_1):
    raise NotImplementedError("write your pallas kernel here")


# speedup vs baseline: 4.8536x; 4.8536x over previous
import jax
import jax.numpy as jnp
from jax import lax
from jax.experimental import pallas as pl
from jax.experimental.pallas import tpu as pltpu
from jax.experimental.pallas import tpu_sc as plsc

NC = 2
NS = 16
NW = NC * NS
LANES = 16
CHUNK = 128


def _make_sc_agg(n_pad, n_chunks, d):
  stride = n_pad // NS
  mesh = plsc.VectorSubcoreMesh(core_axis_name="c", subcore_axis_name="s")

  def body(table_h, src_h, dst_h, agg_out, cnt_out,
           src_v, dst_v, rows_v, ones_v, agg_s, cnt_s, gsem):
    cid = lax.axis_index("c")
    sid = lax.axis_index("s")
    wid = sid * NC + cid

    zero16 = jnp.zeros((LANES,), jnp.float32)
    one16 = jnp.ones((LANES,), jnp.float32)

    def fill(i, carry):
      for k in range(d // LANES):
        rows_v[i, pl.ds(k * LANES, LANES)] = zero16
        ones_v[i, pl.ds(k * LANES, LANES)] = one16
      return carry
    lax.fori_loop(0, CHUNK, fill, 0)

    pltpu.sync_copy(src_h.at[wid], src_v)
    pltpu.sync_copy(dst_h.at[wid], dst_v)

    base = sid * stride
    for k in range(stride // CHUNK):
      pltpu.sync_copy(rows_v, agg_s.at[pl.ds(base + k * CHUNK, CHUNK)])
      pltpu.sync_copy(rows_v, cnt_s.at[pl.ds(base + k * CHUNK, CHUNK)])
    plsc.subcore_barrier()

    def chunk_body(j, carry):
      pltpu.async_copy(table_h.at[src_v.at[j]], rows_v, gsem).wait()
      pltpu.sync_copy(rows_v, agg_s.at[dst_v.at[j]], add=True)
      pltpu.sync_copy(ones_v, cnt_s.at[dst_v.at[j]], add=True)
      return carry
    lax.fori_loop(0, n_chunks, chunk_body, 0)
    plsc.subcore_barrier()

    for k in range(stride // CHUNK):
      pltpu.sync_copy(agg_s.at[pl.ds(base + k * CHUNK, CHUNK)], rows_v)
      pltpu.sync_copy(rows_v, agg_out.at[cid, pl.ds(base + k * CHUNK, CHUNK)])
      pltpu.sync_copy(cnt_s.at[pl.ds(base + k * CHUNK, CHUNK)], ones_v)
      pltpu.sync_copy(ones_v, cnt_out.at[cid, pl.ds(base + k * CHUNK, CHUNK)])

  return pl.kernel(
      body,
      out_type=[
          jax.ShapeDtypeStruct((NC, n_pad, d), jnp.float32),
          jax.ShapeDtypeStruct((NC, n_pad, d), jnp.float32),
      ],
      mesh=mesh,
      scratch_types=[
          pltpu.VMEM((n_chunks, CHUNK), jnp.int32),
          pltpu.VMEM((n_chunks, CHUNK), jnp.int32),
          pltpu.VMEM((CHUNK, d), jnp.float32),
          pltpu.VMEM((CHUNK, d), jnp.float32),
          pltpu.VMEM_SHARED((n_pad, d), jnp.float32),
          pltpu.VMEM_SHARED((n_pad, d), jnp.float32),
          pltpu.SemaphoreType.DMA,
      ],
  )


def _pad_edges(src, dst, n_chunks, dummy_row):
  e = src.shape[0]
  ep = NW * n_chunks * CHUNK
  src_p = jnp.concatenate(
      [src, jnp.zeros((ep - e,), jnp.int32)]).reshape(NW, n_chunks, CHUNK)
  dst_p = jnp.concatenate(
      [dst, jnp.full((ep - e,), dummy_row, jnp.int32)]
  ).reshape(NW, n_chunks, CHUNK)
  return src_p, dst_p


def _dense_tail(x_tgt, agg, cnt, w1t, w2t, b2, gamma, beta, do_bn, d):
  n_tgt = x_tgt.shape[0]

  def body(x_ref, agg_ref, cnt_ref, w1_ref, w2_ref, b2_ref, g_ref, be_ref,
           o_ref):
    agg_sum = agg_ref[0, :n_tgt] + agg_ref[1, :n_tgt]
    cnt_sum = cnt_ref[0, :n_tgt, 0:1] + cnt_ref[1, :n_tgt, 0:1]
    h_n = agg_sum / jnp.maximum(cnt_sum, 1.0)
    out = (jnp.dot(x_ref[...], w1_ref[...],
                   preferred_element_type=jnp.float32)
           + jnp.dot(h_n, w2_ref[...], preferred_element_type=jnp.float32)
           + b2_ref[...])
    nrm = jnp.sqrt(jnp.sum(out * out, axis=1, keepdims=True))
    out = out / jnp.maximum(nrm, 1e-12)
    if do_bn:
      mu = jnp.mean(out, axis=0, keepdims=True)
      var = jnp.mean(out * out, axis=0, keepdims=True) - mu * mu
      out = g_ref[...] * (out - mu) * lax.rsqrt(var + 1e-5) + be_ref[...]
      out = jnp.maximum(out, 0.0)
    o_ref[...] = out

  return pl.pallas_call(
      body,
      out_shape=jax.ShapeDtypeStruct((n_tgt, d), jnp.float32),
  )(x_tgt, agg, cnt, w1t, w2t, b2, gamma, beta)


def kernel(x, src0, dst0, src1, dst1,
           W1_0, W2_0, b2_0, gamma0, beta0,
           W1_1, W2_1, b2_1):
  n0, d = x.shape
  e0 = src0.shape[0]
  e1 = src1.shape[0]
  n1 = 4000
  n2 = 1024
  npad0 = 4096
  npad1 = 2048

  nch0 = -(-e0 // (NW * CHUNK))
  nch1 = -(-e1 // (NW * CHUNK))
  agg_fn0 = _make_sc_agg(npad0, nch0, d)
  agg_fn1 = _make_sc_agg(npad1, nch1, d)

  src0_p, dst0_p = _pad_edges(src0, dst0, nch0, npad0 - 1)
  src1_p, dst1_p = _pad_edges(src1, dst1, nch1, npad1 - 1)

  agg0, cnt0 = agg_fn0(x, src0_p, dst0_p)
  h = _dense_tail(x[:n1], agg0, cnt0, W1_0.T, W2_0.T, b2_0.reshape(1, d),
                  gamma0.reshape(1, d), beta0.reshape(1, d), True, d)
  agg1, cnt1 = agg_fn1(h, src1_p, dst1_p)
  out = _dense_tail(h[:n2], agg1, cnt1, W1_1.T, W2_1.T, b2_1.reshape(1, d),
                    gamma0.reshape(1, d) * 0, beta0.reshape(1, d), False, d)
  return out


# speedup vs baseline: 4.9778x; 1.0256x over previous
import jax
import jax.numpy as jnp
from jax import lax
from jax.experimental import pallas as pl
from jax.experimental.pallas import tpu as pltpu
from jax.experimental.pallas import tpu_sc as plsc

NC = 2
NS = 16
NW = NC * NS
LANES = 16
CHUNK = 128
GROUP = 1


def _make_sc_agg(n_pad, n_chunks, d, pipelined=False):
  stride = n_pad // NS
  mesh = plsc.VectorSubcoreMesh(core_axis_name="c", subcore_axis_name="s")

  def body(table_h, src_h, dst_h, agg_out, cnt_out,
           src_v, dst_v, rows_vv, rows2_v, ones_v, agg_s, cnt_s, gsem, gsem2):
    cid = lax.axis_index("c")
    sid = lax.axis_index("s")
    wid = sid * NC + cid

    zero16 = jnp.zeros((LANES,), jnp.float32)
    one16 = jnp.ones((LANES,), jnp.float32)

    def fill(i, carry):
      for k in range(d // LANES):
        rows_vv[i, pl.ds(k * LANES, LANES)] = zero16
        ones_v[i, pl.ds(k * LANES, LANES)] = one16
      return carry
    lax.fori_loop(0, GROUP * CHUNK, fill, 0)

    pltpu.sync_copy(src_h.at[wid], src_v)
    pltpu.sync_copy(dst_h.at[wid], dst_v)

    base = sid * stride
    blk = min(GROUP * CHUNK, stride)
    for k in range(stride // blk):
      pltpu.sync_copy(rows_vv.at[pl.ds(0, blk)],
                      agg_s.at[pl.ds(base + k * blk, blk)])
      pltpu.sync_copy(rows_vv.at[pl.ds(0, blk)],
                      cnt_s.at[pl.ds(base + k * blk, blk)])
    plsc.subcore_barrier()

    if pipelined:
      pltpu.async_copy(table_h.at[src_v.at[0]], rows2_v, gsem2)

      def ring_body(j, carry):
        par = lax.rem(j, 2)

        @pl.when(par == 0)
        def _():
          pltpu.make_async_copy(table_h.at[src_v.at[j]], rows2_v,
                                gsem2).wait()
          nxt = jnp.minimum(j + 1, n_chunks - 1)
          pltpu.async_copy(table_h.at[src_v.at[nxt]], rows_vv, gsem)
          pltpu.sync_copy(rows2_v, agg_s.at[dst_v.at[j]], add=True)
          pltpu.sync_copy(ones_v, cnt_s.at[dst_v.at[j]], add=True)

        @pl.when(par == 1)
        def _():
          pltpu.make_async_copy(table_h.at[src_v.at[j]], rows_vv,
                                gsem).wait()
          nxt = jnp.minimum(j + 1, n_chunks - 1)
          pltpu.async_copy(table_h.at[src_v.at[nxt]], rows2_v, gsem2)
          pltpu.sync_copy(rows_vv, agg_s.at[dst_v.at[j]], add=True)
          pltpu.sync_copy(ones_v, cnt_s.at[dst_v.at[j]], add=True)
        return carry
      lax.fori_loop(0, n_chunks, ring_body, 0)
      par_end = lax.rem(jnp.int32(n_chunks), 2)

      @pl.when(par_end == 0)
      def _():
        pltpu.make_async_copy(table_h.at[src_v.at[n_chunks - 1]], rows2_v,
                              gsem2).wait()

      @pl.when(par_end == 1)
      def _():
        pltpu.make_async_copy(table_h.at[src_v.at[n_chunks - 1]], rows_vv,
                              gsem).wait()
    else:
      def chunk_body(j, carry):
        idx_g = src_v.at[j]
        idx_s = dst_v.at[j]
        pltpu.async_copy(table_h.at[idx_g], rows_vv, gsem).wait()
        pltpu.sync_copy(rows_vv, agg_s.at[idx_s], add=True)
        pltpu.sync_copy(ones_v, cnt_s.at[idx_s], add=True)
        return carry
      lax.fori_loop(0, n_chunks, chunk_body, 0)
    plsc.subcore_barrier()

    for k in range(stride // blk):
      pltpu.sync_copy(agg_s.at[pl.ds(base + k * blk, blk)],
                      rows_vv.at[pl.ds(0, blk)])
      pltpu.sync_copy(rows_vv.at[pl.ds(0, blk)],
                      agg_out.at[cid, pl.ds(base + k * blk, blk)])
      pltpu.sync_copy(cnt_s.at[pl.ds(base + k * blk, blk)],
                      ones_v.at[pl.ds(0, blk)])
      pltpu.sync_copy(ones_v.at[pl.ds(0, blk)],
                      cnt_out.at[cid, pl.ds(base + k * blk, blk)])

  return pl.kernel(
      body,
      out_type=[
          jax.ShapeDtypeStruct((NC, n_pad, d), jnp.float32),
          jax.ShapeDtypeStruct((NC, n_pad, d), jnp.float32),
      ],
      mesh=mesh,
      scratch_types=[
          pltpu.VMEM((n_chunks, GROUP * CHUNK), jnp.int32),
          pltpu.VMEM((n_chunks, GROUP * CHUNK), jnp.int32),
          pltpu.VMEM((GROUP * CHUNK, d), jnp.float32),
          pltpu.VMEM((GROUP * CHUNK, d), jnp.float32),
          pltpu.VMEM((GROUP * CHUNK, d), jnp.float32),
          pltpu.VMEM_SHARED((n_pad, d), jnp.float32),
          pltpu.VMEM_SHARED((n_pad, d), jnp.float32),
          pltpu.SemaphoreType.DMA,
          pltpu.SemaphoreType.DMA,
      ],
  )


def _pad_edges(src, dst, n_chunks, dummy_row):
  e = src.shape[0]
  ep = NW * n_chunks * GROUP * CHUNK
  src_p = jnp.concatenate(
      [src, jnp.zeros((ep - e,), jnp.int32)]).reshape(NW, n_chunks, GROUP * CHUNK)
  dst_p = jnp.concatenate(
      [dst, jnp.full((ep - e,), dummy_row, jnp.int32)]
  ).reshape(NW, n_chunks, GROUP * CHUNK)
  return src_p, dst_p


def _dense_tail(x_tgt, agg, cnt, w1t, w2t, b2, gamma, beta, do_bn, d,
                pad_row0=0):
  n_tgt = x_tgt.shape[0]

  def body(x_ref, agg_ref, cnt_ref, w1_ref, w2_ref, b2_ref, g_ref, be_ref,
           o_ref):
    agg_sum = agg_ref[0, :n_tgt] + agg_ref[1, :n_tgt]
    cnt_sum = cnt_ref[0, :n_tgt, 0:1] + cnt_ref[1, :n_tgt, 0:1]
    if pad_row0:
      rowmask = (lax.broadcasted_iota(jnp.int32, (n_tgt, 1), 0) == 0)
      corr = jnp.where(rowmask, jnp.float32(pad_row0), 0.0)
      agg_sum = agg_sum - corr * x_ref[0:1, :]
      cnt_sum = cnt_sum - corr
    h_n = agg_sum / jnp.maximum(cnt_sum, 1.0)
    out = (jnp.dot(x_ref[...], w1_ref[...],
                   preferred_element_type=jnp.float32)
           + jnp.dot(h_n, w2_ref[...], preferred_element_type=jnp.float32)
           + b2_ref[...])
    nrm = jnp.sqrt(jnp.sum(out * out, axis=1, keepdims=True))
    out = out / jnp.maximum(nrm, 1e-12)
    if do_bn:
      mu = jnp.mean(out, axis=0, keepdims=True)
      var = jnp.mean(out * out, axis=0, keepdims=True) - mu * mu
      out = g_ref[...] * (out - mu) * lax.rsqrt(var + 1e-5) + be_ref[...]
      out = jnp.maximum(out, 0.0)
    o_ref[...] = out

  return pl.pallas_call(
      body,
      out_shape=jax.ShapeDtypeStruct((n_tgt, d), jnp.float32),
  )(x_tgt, agg, cnt, w1t, w2t, b2, gamma, beta)


def kernel(x, src0, dst0, src1, dst1,
           W1_0, W2_0, b2_0, gamma0, beta0,
           W1_1, W2_1, b2_1):
  n0, d = x.shape
  e0 = src0.shape[0]
  e1 = src1.shape[0]
  n1 = 4000
  n2 = 1024
  npad0 = 4096
  npad1 = 1024

  nch0 = -(-e0 // (NW * GROUP * CHUNK))
  nch1 = -(-e1 // (NW * GROUP * CHUNK))
  agg_fn0 = _make_sc_agg(npad0, nch0, d)
  agg_fn1 = _make_sc_agg(npad1, nch1, d, pipelined=True)

  src0_p, dst0_p = _pad_edges(src0, dst0, nch0, npad0 - 1)
  src1_p, dst1_p = _pad_edges(src1, dst1, nch1, 0)
  pad1 = NW * nch1 * CHUNK - e1

  agg0, cnt0 = agg_fn0(x, src0_p, dst0_p)
  h = _dense_tail(x[:n1], agg0, cnt0, W1_0.T, W2_0.T, b2_0.reshape(1, d),
                  gamma0.reshape(1, d), beta0.reshape(1, d), True, d)
  agg1, cnt1 = agg_fn1(h, src1_p, dst1_p)
  out = _dense_tail(h[:n2], agg1, cnt1, W1_1.T, W2_1.T, b2_1.reshape(1, d),
                    gamma0.reshape(1, d) * 0, beta0.reshape(1, d), False, d,
                    pad_row0=pad1)
  return out


# speedup vs baseline: 5.0032x; 1.0051x over previous
import jax
import jax.numpy as jnp
from jax import lax
from jax.experimental import pallas as pl
from jax.experimental.pallas import tpu as pltpu
from jax.experimental.pallas import tpu_sc as plsc

NC = 2
NS = 16
NW = NC * NS
LANES = 16
CHUNK = 128
GROUP = 1


def _make_sc_agg(n_pad, n_chunks, d, pipelined=False):
  stride = n_pad // NS
  mesh = plsc.VectorSubcoreMesh(core_axis_name="c", subcore_axis_name="s")

  def body(table_h, src_h, dst_h, agg_out, cnt_out, src_v, dst_v, rows_vv,
           rows2_v, ones_v, agg_s, cnt_s, gsem, gsem2, csem):
    cid = lax.axis_index("c")
    sid = lax.axis_index("s")
    wid = sid * NC + cid

    zero16 = jnp.zeros((LANES,), jnp.float32)
    one16 = jnp.ones((LANES,), jnp.float32)

    def fill(i, carry):
      for k in range(d // LANES):
        rows_vv[i, pl.ds(k * LANES, LANES)] = zero16
        ones_v[i, pl.ds(k * LANES, LANES)] = one16
      return carry
    lax.fori_loop(0, GROUP * CHUNK, fill, 0)

    pltpu.sync_copy(src_h.at[wid], src_v)
    pltpu.sync_copy(dst_h.at[wid], dst_v)

    base = sid * stride
    blk = min(GROUP * CHUNK, stride)
    for k in range(stride // blk):
      pltpu.sync_copy(rows_vv.at[pl.ds(0, blk)],
                      agg_s.at[pl.ds(base + k * blk, blk)])
      pltpu.sync_copy(rows_vv.at[pl.ds(0, blk)],
                      cnt_s.at[pl.ds(base + k * blk, blk)])
    plsc.subcore_barrier()

    if pipelined:
      pltpu.async_copy(table_h.at[src_v.at[0]], rows2_v, gsem2)

      def ring_body(j, carry):
        par = lax.rem(j, 2)

        @pl.when(par == 0)
        def _():
          pltpu.make_async_copy(table_h.at[src_v.at[j]], rows2_v,
                                gsem2).wait()
          nxt = jnp.minimum(j + 1, n_chunks - 1)
          pltpu.async_copy(table_h.at[src_v.at[nxt]], rows_vv, gsem)
          cdma = pltpu.async_copy(ones_v, cnt_s.at[dst_v.at[j]], csem,
                                  add=True)
          pltpu.sync_copy(rows2_v, agg_s.at[dst_v.at[j]], add=True)
          cdma.wait()

        @pl.when(par == 1)
        def _():
          pltpu.make_async_copy(table_h.at[src_v.at[j]], rows_vv,
                                gsem).wait()
          nxt = jnp.minimum(j + 1, n_chunks - 1)
          pltpu.async_copy(table_h.at[src_v.at[nxt]], rows2_v, gsem2)
          cdma = pltpu.async_copy(ones_v, cnt_s.at[dst_v.at[j]], csem,
                                  add=True)
          pltpu.sync_copy(rows_vv, agg_s.at[dst_v.at[j]], add=True)
          cdma.wait()
        return carry
      lax.fori_loop(0, n_chunks, ring_body, 0)
      par_end = lax.rem(jnp.int32(n_chunks), 2)

      @pl.when(par_end == 0)
      def _():
        pltpu.make_async_copy(table_h.at[src_v.at[n_chunks - 1]], rows2_v,
                              gsem2).wait()

      @pl.when(par_end == 1)
      def _():
        pltpu.make_async_copy(table_h.at[src_v.at[n_chunks - 1]], rows_vv,
                              gsem).wait()
    else:
      def chunk_body(j, carry):
        idx_g = src_v.at[j]
        idx_s = dst_v.at[j]
        pltpu.async_copy(table_h.at[idx_g], rows_vv, gsem).wait()
        cdma = pltpu.async_copy(ones_v, cnt_s.at[idx_s], csem, add=True)
        pltpu.sync_copy(rows_vv, agg_s.at[idx_s], add=True)
        cdma.wait()
        return carry
      lax.fori_loop(0, n_chunks, chunk_body, 0)
    plsc.subcore_barrier()

    for k in range(stride // blk):
      pltpu.sync_copy(agg_s.at[pl.ds(base + k * blk, blk)],
                      rows_vv.at[pl.ds(0, blk)])
      pltpu.sync_copy(rows_vv.at[pl.ds(0, blk)],
                      agg_out.at[cid, pl.ds(base + k * blk, blk)])
      pltpu.sync_copy(cnt_s.at[pl.ds(base + k * blk, blk)],
                      ones_v.at[pl.ds(0, blk)])
      pltpu.sync_copy(ones_v.at[pl.ds(0, blk)],
                      cnt_out.at[cid, pl.ds(base + k * blk, blk)])

  return pl.kernel(
      body,
      out_type=[
          jax.ShapeDtypeStruct((NC, n_pad, d), jnp.float32),
          jax.ShapeDtypeStruct((NC, n_pad, d), jnp.float32),
      ],
      mesh=mesh,
      scratch_types=[
          pltpu.VMEM((n_chunks, GROUP * CHUNK), jnp.int32),
          pltpu.VMEM((n_chunks, GROUP * CHUNK), jnp.int32),
          pltpu.VMEM((GROUP * CHUNK, d), jnp.float32),
          pltpu.VMEM((GROUP * CHUNK, d), jnp.float32),
          pltpu.VMEM((GROUP * CHUNK, d), jnp.float32),
          pltpu.VMEM_SHARED((n_pad, d), jnp.float32),
          pltpu.VMEM_SHARED((n_pad, d), jnp.float32),
          pltpu.SemaphoreType.DMA,
          pltpu.SemaphoreType.DMA,
          pltpu.SemaphoreType.DMA,
      ],
  )


def _pad_tc(arr, pad_rows, value):
  r = arr.shape[0]

  def body(x_ref, o_ref):
    tail = jnp.full((pad_rows, CHUNK), value, jnp.int32)
    o_ref[...] = jnp.concatenate([x_ref[...], tail], axis=0)

  return pl.pallas_call(
      body,
      out_shape=jax.ShapeDtypeStruct((r + pad_rows, CHUNK), jnp.int32),
  )(arr)


def _pad_edges(src, dst, n_chunks, dummy_row):
  e = src.shape[0]
  ep = NW * n_chunks * GROUP * CHUNK
  pad_rows = (ep - e) // CHUNK
  src_p = _pad_tc(src.reshape(-1, CHUNK), pad_rows, 0)
  dst_p = _pad_tc(dst.reshape(-1, CHUNK), pad_rows, dummy_row)
  return (src_p.reshape(NW, n_chunks, GROUP * CHUNK),
          dst_p.reshape(NW, n_chunks, GROUP * CHUNK))


def _dense_tail(x_tgt, agg, cnt, w1t, w2t, b2, gamma, beta, do_bn, d,
                pad_row0=0):
  n_tgt = x_tgt.shape[0]

  def body(x_ref, agg_ref, cnt_ref, w1_ref, w2_ref, b2_ref, g_ref, be_ref,
           o_ref):
    agg_sum = agg_ref[0, :n_tgt] + agg_ref[1, :n_tgt]
    cnt_sum = cnt_ref[0, :n_tgt, 0:1] + cnt_ref[1, :n_tgt, 0:1]
    if pad_row0:
      rowmask = (lax.broadcasted_iota(jnp.int32, (n_tgt, 1), 0) == 0)
      corr = jnp.where(rowmask, jnp.float32(pad_row0), 0.0)
      agg_sum = agg_sum - corr * x_ref[0:1, :]
      cnt_sum = cnt_sum - corr
    h_n = agg_sum / jnp.maximum(cnt_sum, 1.0)
    out = (jnp.dot(x_ref[...], w1_ref[...],
                   preferred_element_type=jnp.float32)
           + jnp.dot(h_n, w2_ref[...], preferred_element_type=jnp.float32)
           + b2_ref[...])
    nrm = jnp.sqrt(jnp.sum(out * out, axis=1, keepdims=True))
    out = out / jnp.maximum(nrm, 1e-12)
    if do_bn:
      mu = jnp.mean(out, axis=0, keepdims=True)
      var = jnp.mean(out * out, axis=0, keepdims=True) - mu * mu
      out = g_ref[...] * (out - mu) * lax.rsqrt(var + 1e-5) + be_ref[...]
      out = jnp.maximum(out, 0.0)
    o_ref[...] = out

  return pl.pallas_call(
      body,
      out_shape=jax.ShapeDtypeStruct((n_tgt, d), jnp.float32),
  )(x_tgt, agg, cnt, w1t, w2t, b2, gamma, beta)


def kernel(x, src0, dst0, src1, dst1,
           W1_0, W2_0, b2_0, gamma0, beta0,
           W1_1, W2_1, b2_1):
  n0, d = x.shape
  e0 = src0.shape[0]
  e1 = src1.shape[0]
  n1 = 4000
  n2 = 1024
  npad0 = 4096
  npad1 = 1024

  nch0 = -(-e0 // (NW * GROUP * CHUNK))
  nch1 = -(-e1 // (NW * GROUP * CHUNK))
  agg_fn0 = _make_sc_agg(npad0, nch0, d)
  agg_fn1 = _make_sc_agg(npad1, nch1, d, pipelined=True)

  src0_p, dst0_p = _pad_edges(src0, dst0, nch0, npad0 - 1)
  src1_p, dst1_p = _pad_edges(src1, dst1, nch1, 0)
  pad1 = NW * nch1 * CHUNK - e1

  agg0, cnt0 = agg_fn0(x, src0_p, dst0_p)
  h = _dense_tail(x[:n1], agg0, cnt0, W1_0.T, W2_0.T, b2_0.reshape(1, d),
                  gamma0.reshape(1, d), beta0.reshape(1, d), True, d)
  agg1, cnt1 = agg_fn1(h, src1_p, dst1_p)
  out = _dense_tail(h[:n2], agg1, cnt1, W1_1.T, W2_1.T, b2_1.reshape(1, d),
                    gamma0.reshape(1, d) * 0, beta0.reshape(1, d), False, d,
                    pad_row0=pad1)
  return out


# speedup vs baseline: 5.0138x; 1.0021x over previous
import jax
import jax.numpy as jnp
from jax import lax
from jax.experimental import pallas as pl
from jax.experimental.pallas import tpu as pltpu
from jax.experimental.pallas import tpu_sc as plsc

NC = 2
NS = 16
NW = NC * NS
LANES = 16
CHUNK = 128
GROUP = 1


def _make_sc_agg(n_pad, n_chunks, d, pipelined=False):
  stride = n_pad // NS
  mesh = plsc.VectorSubcoreMesh(core_axis_name="c", subcore_axis_name="s")

  def body(table_h, src_h, dst_h, agg_out, cnt_out, src_v, dst_v, rows_vv,
           rows2_v, ones_v, agg_s, cnt_s, gsem, gsem2, csem):
    cid = lax.axis_index("c")
    sid = lax.axis_index("s")
    wid = sid * NC + cid

    zero16 = jnp.zeros((LANES,), jnp.float32)
    one16 = jnp.ones((LANES,), jnp.float32)

    def fill(i, carry):
      for k in range(d // LANES):
        rows_vv[i, pl.ds(k * LANES, LANES)] = zero16
        ones_v[i, pl.ds(k * LANES, LANES)] = one16
      return carry
    lax.fori_loop(0, GROUP * CHUNK, fill, 0)

    pltpu.sync_copy(src_h.at[wid], src_v)
    pltpu.sync_copy(dst_h.at[wid], dst_v)

    base = sid * stride
    blk = min(GROUP * CHUNK, stride)
    for k in range(stride // blk):
      pltpu.sync_copy(rows_vv.at[pl.ds(0, blk)],
                      agg_s.at[pl.ds(base + k * blk, blk)])
      pltpu.sync_copy(rows_vv.at[pl.ds(0, blk)],
                      cnt_s.at[pl.ds(base + k * blk, blk)])
    plsc.subcore_barrier()

    if pipelined:
      pltpu.async_copy(table_h.at[src_v.at[0]], rows2_v, gsem2)

      def ring_body(j, carry):
        par = lax.rem(j, 2)

        @pl.when(par == 0)
        def _():
          pltpu.make_async_copy(table_h.at[src_v.at[j]], rows2_v,
                                gsem2).wait()
          nxt = jnp.minimum(j + 1, n_chunks - 1)
          pltpu.async_copy(table_h.at[src_v.at[nxt]], rows_vv, gsem)
          cdma = pltpu.async_copy(ones_v, cnt_s.at[dst_v.at[j]], csem,
                                  add=True)
          pltpu.sync_copy(rows2_v, agg_s.at[dst_v.at[j]], add=True)
          cdma.wait()

        @pl.when(par == 1)
        def _():
          pltpu.make_async_copy(table_h.at[src_v.at[j]], rows_vv,
                                gsem).wait()
          nxt = jnp.minimum(j + 1, n_chunks - 1)
          pltpu.async_copy(table_h.at[src_v.at[nxt]], rows2_v, gsem2)
          cdma = pltpu.async_copy(ones_v, cnt_s.at[dst_v.at[j]], csem,
                                  add=True)
          pltpu.sync_copy(rows_vv, agg_s.at[dst_v.at[j]], add=True)
          cdma.wait()
        return carry
      lax.fori_loop(0, n_chunks, ring_body, 0)
      par_end = lax.rem(jnp.int32(n_chunks), 2)

      @pl.when(par_end == 0)
      def _():
        pltpu.make_async_copy(table_h.at[src_v.at[n_chunks - 1]], rows2_v,
                              gsem2).wait()

      @pl.when(par_end == 1)
      def _():
        pltpu.make_async_copy(table_h.at[src_v.at[n_chunks - 1]], rows_vv,
                              gsem).wait()
    else:
      def chunk_body(j, carry):
        idx_g = src_v.at[j]
        idx_s = dst_v.at[j]
        pltpu.async_copy(table_h.at[idx_g], rows_vv, gsem).wait()
        cdma = pltpu.async_copy(ones_v, cnt_s.at[idx_s], csem, add=True)
        pltpu.sync_copy(rows_vv, agg_s.at[idx_s], add=True)
        cdma.wait()
        return carry
      lax.fori_loop(0, n_chunks, chunk_body, 0)
    plsc.subcore_barrier()

    for k in range(stride // blk):
      pltpu.sync_copy(agg_s.at[pl.ds(base + k * blk, blk)],
                      rows_vv.at[pl.ds(0, blk)])
      pltpu.sync_copy(rows_vv.at[pl.ds(0, blk)],
                      agg_out.at[cid, pl.ds(base + k * blk, blk)])
      pltpu.sync_copy(cnt_s.at[pl.ds(base + k * blk, blk)],
                      ones_v.at[pl.ds(0, blk)])
      pltpu.sync_copy(ones_v.at[pl.ds(0, blk)],
                      cnt_out.at[cid, pl.ds(base + k * blk, blk)])

  return pl.kernel(
      body,
      out_type=[
          jax.ShapeDtypeStruct((NC, n_pad, d), jnp.float32),
          jax.ShapeDtypeStruct((NC, n_pad, d), jnp.float32),
      ],
      mesh=mesh,
      scratch_types=[
          pltpu.VMEM((n_chunks, GROUP * CHUNK), jnp.int32),
          pltpu.VMEM((n_chunks, GROUP * CHUNK), jnp.int32),
          pltpu.VMEM((GROUP * CHUNK, d), jnp.float32),
          pltpu.VMEM((GROUP * CHUNK, d), jnp.float32),
          pltpu.VMEM((GROUP * CHUNK, d), jnp.float32),
          pltpu.VMEM_SHARED((n_pad, d), jnp.float32),
          pltpu.VMEM_SHARED((n_pad, d), jnp.float32),
          pltpu.SemaphoreType.DMA,
          pltpu.SemaphoreType.DMA,
          pltpu.SemaphoreType.DMA,
      ],
  )


def _pad_edges(src, dst, n_chunks, dummy_row):
  e = src.shape[0]
  ep = NW * n_chunks * GROUP * CHUNK
  src_p = jnp.concatenate(
      [src, jnp.zeros((ep - e,), jnp.int32)]).reshape(NW, n_chunks, GROUP * CHUNK)
  dst_p = jnp.concatenate(
      [dst, jnp.full((ep - e,), dummy_row, jnp.int32)]
  ).reshape(NW, n_chunks, GROUP * CHUNK)
  return src_p, dst_p


def _dense_tail(x_tgt, agg, cnt, w1t, w2t, b2, gamma, beta, do_bn, d,
                pad_row0=0):
  n_tgt = x_tgt.shape[0]

  def body(x_ref, agg_ref, cnt_ref, w1_ref, w2_ref, b2_ref, g_ref, be_ref,
           o_ref):
    agg_sum = agg_ref[0, :n_tgt] + agg_ref[1, :n_tgt]
    cnt_sum = cnt_ref[0, :n_tgt, 0:1] + cnt_ref[1, :n_tgt, 0:1]
    if pad_row0:
      rowmask = (lax.broadcasted_iota(jnp.int32, (n_tgt, 1), 0) == 0)
      corr = jnp.where(rowmask, jnp.float32(pad_row0), 0.0)
      agg_sum = agg_sum - corr * x_ref[0:1, :]
      cnt_sum = cnt_sum - corr
    h_n = agg_sum / jnp.maximum(cnt_sum, 1.0)
    out = (jnp.dot(x_ref[...], w1_ref[...],
                   preferred_element_type=jnp.float32)
           + jnp.dot(h_n, w2_ref[...], preferred_element_type=jnp.float32)
           + b2_ref[...])
    nrm = jnp.sqrt(jnp.sum(out * out, axis=1, keepdims=True))
    out = out / jnp.maximum(nrm, 1e-12)
    if do_bn:
      mu = jnp.mean(out, axis=0, keepdims=True)
      var = jnp.mean(out * out, axis=0, keepdims=True) - mu * mu
      out = g_ref[...] * (out - mu) * lax.rsqrt(var + 1e-5) + be_ref[...]
      out = jnp.maximum(out, 0.0)
    o_ref[...] = out

  return pl.pallas_call(
      body,
      out_shape=jax.ShapeDtypeStruct((n_tgt, d), jnp.float32),
  )(x_tgt, agg, cnt, w1t, w2t, b2, gamma, beta)


def kernel(x, src0, dst0, src1, dst1,
           W1_0, W2_0, b2_0, gamma0, beta0,
           W1_1, W2_1, b2_1):
  n0, d = x.shape
  e0 = src0.shape[0]
  e1 = src1.shape[0]
  n1 = 4000
  n2 = 1024
  npad0 = 4096
  npad1 = 1024

  nch0 = -(-e0 // (NW * GROUP * CHUNK))
  nch1 = -(-e1 // (NW * GROUP * CHUNK))
  agg_fn0 = _make_sc_agg(npad0, nch0, d)
  agg_fn1 = _make_sc_agg(npad1, nch1, d, pipelined=True)

  src0_p, dst0_p = _pad_edges(src0, dst0, nch0, npad0 - 1)
  src1_p, dst1_p = _pad_edges(src1, dst1, nch1, 0)
  pad1 = NW * nch1 * CHUNK - e1

  agg0, cnt0 = agg_fn0(x, src0_p, dst0_p)
  h = _dense_tail(x[:n1], agg0, cnt0, W1_0.T, W2_0.T, b2_0.reshape(1, d),
                  gamma0.reshape(1, d), beta0.reshape(1, d), True, d)
  agg1, cnt1 = agg_fn1(h, src1_p, dst1_p)
  out = _dense_tail(h[:n2], agg1, cnt1, W1_1.T, W2_1.T, b2_1.reshape(1, d),
                    gamma0.reshape(1, d) * 0, beta0.reshape(1, d), False, d,
                    pad_row0=pad1)
  return out
